# baseline (speedup 1.0000x reference)
import functools

import jax
import jax.numpy as jnp
from jax import lax
from jax.experimental import pallas as pl
from jax.experimental.pallas import tpu as pltpu
from jax.experimental.pallas import tpu_sc as plsc

S = 32
L = 1024
K = 1000
D = 64
TOPK = 64

F32 = jnp.float32
I32 = jnp.int32


def _encode_body(f_ref, cbt_ref, out_ref):
    f = f_ref[...]
    cbt = cbt_ref[...]
    prod = jnp.dot(f, cbt, preferred_element_type=F32)
    c2 = jnp.sum(cbt * cbt, axis=0, keepdims=True)
    f2 = jnp.sum(f * f, axis=1, keepdims=True)
    d2 = f2 - 2.0 * prod + c2
    codes = jnp.argmin(d2, axis=-1).astype(I32)
    out_ref[...] = codes.reshape(1, 1, 128)


def _encode(frames, cbT):
    out = pl.pallas_call(
        _encode_body,
        grid=(8,),
        in_specs=[
            pl.BlockSpec((128, D), lambda i: (i, 0)),
            pl.BlockSpec((D, K), lambda i: (0, 0)),
        ],
        out_specs=pl.BlockSpec((1, 1, 128), lambda i: (i, 0, 0)),
        out_shape=jax.ShapeDtypeStruct((8, 1, 128), I32),
    )(frames, cbT)
    return out.reshape(L)


def _kth_body(lg_ref, out_ref):
    x = lg_ref[...]
    bits = jax.lax.bitcast_convert_type(x, I32)
    MIN = jnp.int32(-(2**31))
    keys = jnp.where(bits >= 0, bits | MIN, ~bits)
    keysx = keys ^ MIN
    t = jnp.zeros((128, 1), I32)
    for b in range(31, -1, -1):
        bit = MIN if b == 31 else jnp.int32(1 << b)
        cand = t | bit
        candx = cand ^ MIN
        cnt = jnp.sum((keysx >= candx).astype(I32), axis=1, keepdims=True)
        t = jnp.where(cnt >= TOPK, cand, t)
    fbits = jnp.where(t < 0, t & jnp.int32(2**31 - 1), ~t)
    kth = jax.lax.bitcast_convert_type(fbits, F32)
    out_ref[...] = jnp.broadcast_to(kth, (128, 128))


def _kth(logits):
    return pl.pallas_call(
        _kth_body,
        grid=(8,),
        in_specs=[pl.BlockSpec((128, KP), lambda i: (i, 0))],
        out_specs=pl.BlockSpec((128, 128), lambda i: (i, 0)),
        out_shape=jax.ShapeDtypeStruct((L, 128), F32),
    )(logits)


def _sample_body(u_ref, lg_ref, kth_ref, out_ref):
    u = u_ref[...]
    lg = lg_ref[...][:, :K]
    kth = kth_ref[...][:, :1]
    filt = jnp.where(lg >= kth, lg, -jnp.inf)
    g = -jnp.log(-jnp.log(u + 1e-9) + 1e-9)
    x = jnp.argmax(filt[None] + g, axis=-1).astype(I32)
    out_ref[...] = x.reshape(1, 1, 8, 128)


def _sample(u, logits, kth_b):
    x4d = pl.pallas_call(
        _sample_body,
        grid=(4, 8),
        in_specs=[
            pl.BlockSpec((8, 128, K), lambda sb, lg: (sb, lg, 0)),
            pl.BlockSpec((128, KP), lambda sb, lg: (lg, 0)),
            pl.BlockSpec((128, 128), lambda sb, lg: (lg, 0)),
        ],
        out_specs=pl.BlockSpec((1, 1, 8, 128), lambda sb, lg: (sb, lg, 0, 0)),
        out_shape=jax.ShapeDtypeStruct((4, 8, 8, 128), I32),
    )(u, logits, kth_b)
    return x4d.transpose(0, 2, 1, 3).reshape(S, L)


NW = 32
_SC_MESH = plsc.VectorSubcoreMesh(core_axis_name="c", subcore_axis_name="s")

KP = 1024


@functools.partial(
    pl.kernel, mesh=_SC_MESH,
    out_type=[jax.ShapeDtypeStruct((L, KP), F32),
              jax.ShapeDtypeStruct((L, KP), F32)],
    scratch_types=[pltpu.VMEM((L // NW,), I32),
                   pltpu.VMEM((L // NW, KP), F32),
                   pltpu.SemaphoreType.DMA],
)
def _gather_priors(t0_hbm, t1_hbm, codes_hbm, o0_hbm, o1_hbm, idx_v, rows_v, sem):
    bpw = L // NW
    wid = lax.axis_index("s") * 2 + lax.axis_index("c")
    base = wid * bpw
    pltpu.sync_copy(codes_hbm.at[pl.ds(base, bpw)], idx_v)
    pltpu.async_copy(t0_hbm.at[idx_v], rows_v, sem).wait()
    pltpu.sync_copy(rows_v, o0_hbm.at[pl.ds(base, bpw)])
    pltpu.async_copy(t1_hbm.at[idx_v], rows_v, sem).wait()
    pltpu.sync_copy(rows_v, o1_hbm.at[pl.ds(base, bpw)])


_G2_CHUNK = 128


DP = 128


@functools.partial(
    pl.kernel, mesh=_SC_MESH,
    out_type=[jax.ShapeDtypeStruct((S * L, DP), F32),
              jax.ShapeDtypeStruct((S * L, DP), F32)],
    scratch_types=[pltpu.VMEM((S * L // NW,), I32),
                   pltpu.VMEM((_G2_CHUNK, DP), F32),
                   pltpu.SemaphoreType.DMA],
)
def _gather_codebook(cb_hbm, x0_hbm, x1_hbm, o0_hbm, o1_hbm, idx_v, rows_v, sem):
    bpw = S * L // NW
    wid = lax.axis_index("s") * 2 + lax.axis_index("c")
    base = wid * bpw
    for x_hbm, o_hbm in ((x0_hbm, o0_hbm), (x1_hbm, o1_hbm)):
        pltpu.sync_copy(x_hbm.at[pl.ds(base, bpw)], idx_v)
        for c in range(bpw // _G2_CHUNK):
            pltpu.async_copy(
                cb_hbm.at[idx_v.at[pl.ds(c * _G2_CHUNK, _G2_CHUNK)]],
                rows_v, sem).wait()
            pltpu.sync_copy(rows_v, o_hbm.at[pl.ds(base + c * _G2_CHUNK, _G2_CHUNK)])


def _select_body(r0_ref, r1_ref, mix_ref, out_ref, err_ref):
    i = pl.program_id(0)
    s = i % S
    r0 = r0_ref[...][:, :D]
    r1 = r1_ref[...][:, :D]
    m = mix_ref[...]

    @pl.when(i < S)
    def _():
        diff = 0.5 * r0 + 0.5 * r1 - m
        err_ref[s] = jnp.sqrt(jnp.sum(diff * diff))

    @pl.when(i >= S)
    def _():
        def body(j, carry):
            bv, bi = carry
            v = err_ref[j]
            take = v < bv
            return (jnp.where(take, v, bv), jnp.where(take, j, bi))
        _, best = jax.lax.fori_loop(0, S, body, (jnp.float32(jnp.inf), jnp.int32(0)))

        @pl.when(s == best)
        def _():
            out_ref[...] = jnp.concatenate([r0[None], r1[None]], axis=0)


def _select(res0, res1, mix2d):
    return pl.pallas_call(
        _select_body,
        grid=(2 * S,),
        in_specs=[
            pl.BlockSpec((L, DP), lambda i: (i % S, 0)),
            pl.BlockSpec((L, DP), lambda i: (i % S, 0)),
            pl.BlockSpec((L, D), lambda i: (0, 0)),
        ],
        out_specs=pl.BlockSpec((2, L, D), lambda i: (0, 0, 0)),
        out_shape=jax.ShapeDtypeStruct((2, L, D), F32),
        scratch_shapes=[pltpu.SMEM((S,), F32)],
    )(res0, res1, mix2d)


def kernel(mixture, u0, u1, codebook, prior0, prior1):
    frames = mixture.reshape(L, D)
    cbT = codebook.T

    codes = _encode(frames, cbT)

    p0 = jnp.pad(prior0, ((0, 0), (0, KP - K)), constant_values=-jnp.inf)
    p1 = jnp.pad(prior1, ((0, 0), (0, KP - K)), constant_values=-jnp.inf)
    logits0, logits1 = _gather_priors(p0, p1, codes)

    kth0 = _kth(logits0)
    kth1 = _kth(logits1)

    x0 = jnp.zeros((S, L), I32) + codes[None, :] * 0 + jnp.float32(0).astype(I32)
    x1 = jnp.zeros((S, L), I32)
    kth0sum = kth0.sum() + kth1.sum()
    x0 = x0 + (kth0sum * 0).astype(I32)

    cbp = jnp.pad(codebook, ((0, 0), (0, DP - D)))
    res0, res1 = _gather_codebook(cbp, x0.reshape(-1), x1.reshape(-1))

    out = _select(res0, res1, frames)
    return out.reshape(2, L * D)


# speedup vs baseline: 4.8607x; 4.8607x over previous
import functools

import jax
import jax.numpy as jnp
from jax import lax
from jax.experimental import pallas as pl
from jax.experimental.pallas import tpu as pltpu
from jax.experimental.pallas import tpu_sc as plsc

S = 32
L = 1024
K = 1000
D = 64
TOPK = 64

F32 = jnp.float32
I32 = jnp.int32


def _encode_body(f_ref, cbt_ref, out_ref):
    f = f_ref[...]
    cbt = cbt_ref[...]
    prod = jnp.dot(f, cbt, preferred_element_type=F32)
    c2 = jnp.sum(cbt * cbt, axis=0, keepdims=True)
    f2 = jnp.sum(f * f, axis=1, keepdims=True)
    d2 = f2 - 2.0 * prod + c2
    codes = jnp.argmin(d2, axis=-1).astype(I32)
    out_ref[...] = codes.reshape(1, 1, 128)


def _encode(frames, cbT):
    out = pl.pallas_call(
        _encode_body,
        grid=(8,),
        in_specs=[
            pl.BlockSpec((128, D), lambda i: (i, 0)),
            pl.BlockSpec((D, K), lambda i: (0, 0)),
        ],
        out_specs=pl.BlockSpec((1, 1, 128), lambda i: (i, 0, 0)),
        out_shape=jax.ShapeDtypeStruct((8, 1, 128), I32),
    )(frames, cbT)
    return out.reshape(L)


def _kth_body(lg_ref, out_ref):
    x = lg_ref[...]
    R = x.shape[0]
    bits = jax.lax.bitcast_convert_type(x, I32)
    MIN = jnp.int32(-(2**31))
    keys = jnp.where(bits >= 0, bits | MIN, ~bits)
    keysx = keys ^ MIN
    t = jnp.zeros((R, 1), I32)
    for b in range(31, -1, -1):
        bit = MIN if b == 31 else jnp.int32(1 << b)
        cand = t | bit
        candx = cand ^ MIN
        cnt = jnp.sum((keysx >= candx).astype(I32), axis=1, keepdims=True)
        t = jnp.where(cnt >= TOPK, cand, t)
    fbits = jnp.where(t < 0, t & jnp.int32(2**31 - 1), ~t)
    kth = jax.lax.bitcast_convert_type(fbits, F32)
    out_ref[...] = jnp.broadcast_to(kth, (R, 128))


def _kth(logits):
    return pl.pallas_call(
        _kth_body,
        grid=(1,),
        in_specs=[pl.BlockSpec((L, KP), lambda i: (0, 0))],
        out_specs=pl.BlockSpec((L, 128), lambda i: (0, 0)),
        out_shape=jax.ShapeDtypeStruct((L, 128), F32),
    )(logits)


def _sample_body(u_ref, lg_ref, kth_ref, out_ref):
    u = u_ref[...]
    lg = lg_ref[...][:, :K]
    kth = kth_ref[...][:, :1]
    filt = jnp.where(lg >= kth, lg, -jnp.inf)
    g = -jnp.log(-jnp.log(u + 1e-9) + 1e-9)
    x = jnp.argmax(filt[None] + g, axis=-1).astype(I32)
    out_ref[...] = x.reshape(1, 1, 8, 128)


def _sample(u, logits, kth_b):
    x4d = pl.pallas_call(
        _sample_body,
        grid=(8, 4),
        in_specs=[
            pl.BlockSpec((8, 128, K), lambda lg, sb: (sb, lg, 0)),
            pl.BlockSpec((128, KP), lambda lg, sb: (lg, 0)),
            pl.BlockSpec((128, 128), lambda lg, sb: (lg, 0)),
        ],
        out_specs=pl.BlockSpec((1, 1, 8, 128), lambda lg, sb: (sb, lg, 0, 0)),
        out_shape=jax.ShapeDtypeStruct((4, 8, 8, 128), I32),
    )(u, logits, kth_b)
    return x4d.transpose(0, 2, 1, 3).reshape(S, L)


NW = 32
_SC_MESH = plsc.VectorSubcoreMesh(core_axis_name="c", subcore_axis_name="s")

KP = 1024


@functools.partial(
    pl.kernel, mesh=_SC_MESH,
    out_type=[jax.ShapeDtypeStruct((L, KP), F32),
              jax.ShapeDtypeStruct((L, KP), F32)],
    scratch_types=[pltpu.VMEM((L // NW,), I32),
                   pltpu.VMEM((L // NW, KP), F32),
                   pltpu.SemaphoreType.DMA],
)
def _gather_priors(t0_hbm, t1_hbm, codes_hbm, o0_hbm, o1_hbm, idx_v, rows_v, sem):
    bpw = L // NW
    wid = lax.axis_index("s") * 2 + lax.axis_index("c")
    base = wid * bpw
    pltpu.sync_copy(codes_hbm.at[pl.ds(base, bpw)], idx_v)
    pltpu.async_copy(t0_hbm.at[idx_v], rows_v, sem).wait()
    pltpu.sync_copy(rows_v, o0_hbm.at[pl.ds(base, bpw)])
    pltpu.async_copy(t1_hbm.at[idx_v], rows_v, sem).wait()
    pltpu.sync_copy(rows_v, o1_hbm.at[pl.ds(base, bpw)])


_G2_CHUNK = 128


DP = 128


@functools.partial(
    pl.kernel, mesh=_SC_MESH,
    out_type=jax.ShapeDtypeStruct((S * L, DP), F32),
    scratch_types=[pltpu.VMEM((S * L // NW,), I32),
                   pltpu.VMEM((_G2_CHUNK, DP), F32),
                   pltpu.VMEM((_G2_CHUNK, DP), F32),
                   pltpu.SemaphoreType.DMA,
                   pltpu.SemaphoreType.DMA],
)
def _gather_codebook(cb_hbm, x_hbm, o_hbm, idx_v, rows_a, rows_b, sem_a, sem_b):
    bpw = S * L // NW
    wid = lax.axis_index("s") * 2 + lax.axis_index("c")
    base = wid * bpw
    nchunk = bpw // _G2_CHUNK
    pltpu.sync_copy(x_hbm.at[pl.ds(base, bpw)], idx_v)
    bufs = ((rows_a, sem_a), (rows_b, sem_b))
    cps = [pltpu.async_copy(cb_hbm.at[idx_v.at[pl.ds(0, _G2_CHUNK)]],
                            rows_a, sem_a)]
    for c in range(nchunk):
        if c + 1 < nchunk:
            rows_n, sem_n = bufs[(c + 1) % 2]
            cps.append(pltpu.async_copy(
                cb_hbm.at[idx_v.at[pl.ds((c + 1) * _G2_CHUNK, _G2_CHUNK)]],
                rows_n, sem_n))
        rows_c, _ = bufs[c % 2]
        cps[c].wait()
        pltpu.sync_copy(rows_c, o_hbm.at[pl.ds(base + c * _G2_CHUNK, _G2_CHUNK)])


def _select_body(r0_ref, r1_ref, mix_ref, out_ref, err_ref):
    i = pl.program_id(0)
    s = i % S
    r0 = r0_ref[...][:, :D]
    r1 = r1_ref[...][:, :D]
    m = mix_ref[...]

    @pl.when(i < S)
    def _():
        diff = 0.5 * r0 + 0.5 * r1 - m
        err_ref[s] = jnp.sqrt(jnp.sum(diff * diff))

    @pl.when(i >= S)
    def _():
        def body(j, carry):
            bv, bi = carry
            v = err_ref[j]
            take = v < bv
            return (jnp.where(take, v, bv), jnp.where(take, j, bi))
        _, best = jax.lax.fori_loop(0, S, body, (jnp.float32(jnp.inf), jnp.int32(0)))

        @pl.when(s == best)
        def _():
            out_ref[...] = jnp.concatenate([r0[None], r1[None]], axis=0)


def _select(res0, res1, mix2d):
    return pl.pallas_call(
        _select_body,
        grid=(2 * S,),
        in_specs=[
            pl.BlockSpec((L, DP), lambda i: (i % S, 0)),
            pl.BlockSpec((L, DP), lambda i: (i % S, 0)),
            pl.BlockSpec((L, D), lambda i: (0, 0)),
        ],
        out_specs=pl.BlockSpec((2, L, D), lambda i: (0, 0, 0)),
        out_shape=jax.ShapeDtypeStruct((2, L, D), F32),
        scratch_shapes=[pltpu.SMEM((S,), F32)],
    )(res0, res1, mix2d)


def kernel(mixture, u0, u1, codebook, prior0, prior1):
    frames = mixture.reshape(L, D)
    cbT = codebook.T

    codes = _encode(frames, cbT)

    p0 = jnp.pad(prior0, ((0, 0), (0, KP - K)), constant_values=-jnp.inf)
    p1 = jnp.pad(prior1, ((0, 0), (0, KP - K)), constant_values=-jnp.inf)
    logits0, logits1 = _gather_priors(p0, p1, codes)

    kth0 = _kth(logits0)
    kth1 = _kth(logits1)

    x0 = _sample(u0, logits0, kth0)
    x1 = _sample(u1, logits1, kth1)

    cbp = jnp.pad(codebook, ((0, 0), (0, DP - D)))
    res0 = _gather_codebook(cbp, x0.reshape(-1))
    res1 = _gather_codebook(cbp, x1.reshape(-1))

    out = _select(res0, res1, frames)
    return out.reshape(2, L * D)
